# baseline (speedup 1.0000x reference)
import functools

import jax
import jax.numpy as jnp
from jax import lax
from jax.experimental import pallas as pl
from jax.experimental.pallas import tpu as pltpu
from jax.experimental.pallas import tpu_sc as plsc

_NC = 2
_NS = 16
_NW = _NC * _NS
_L = 16

_ROWS = 48
_ROW = 512 * 512
_TOTAL = _ROWS * _ROW
_CHUNK = 8192
_CHUNKS_PER_W = _TOTAL // _NW // _CHUNK
_PER_W = _TOTAL // _NW
_HIST = 2 * 2 * _L * 256
_OUT = 2 * _NW * 2 * 256


def _sc_body(h1_hbm, h2_hbm, out_hbm, buf, hist, stage,
             sem0, sem1, sem2, sem3):
    wid = lax.axis_index("s") * _NC + lax.axis_index("c")
    base_w = wid * _PER_W
    sems = (sem0, sem1, sem2, sem3)
    lane_vec = lax.iota(jnp.int32, _L) * 256
    ones = jnp.full((_L,), 1.0, jnp.float32)
    zeros = jnp.zeros((_L,), jnp.float32)

    @plsc.parallel_loop(0, _HIST, _L, unroll=4)
    def _(i):
        hist[pl.ds(i, _L)] = zeros

    def issue(j, slot):
        row0 = pl.multiple_of((base_w + j * _CHUNK) // 512, _CHUNK // 512)
        pltpu.async_copy(h1_hbm.at[pl.ds(row0, _CHUNK // 512)], buf.at[0, slot],
                         sems[slot])
        pltpu.async_copy(h2_hbm.at[pl.ds(row0, _CHUNK // 512)], buf.at[1, slot],
                         sems[slot])

    def wait(slot):
        pltpu.make_async_copy(
            h1_hbm.at[pl.ds(0, _CHUNK // 512)], buf.at[0, slot], sems[slot]).wait()
        pltpu.make_async_copy(
            h2_hbm.at[pl.ds(0, _CHUNK // 512)], buf.at[1, slot], sems[slot]).wait()

    def process(j, slot):
        cpr = _ROW // _CHUNK
        l = (j + (cpr // 2) * (wid % 2)) // cpr
        for a in (0, 1):
            base_vec = lane_vec + a * (2 * _L * 256) + l * (_L * 256)

            @plsc.parallel_loop(0, _CHUNK // _L, 1, unroll=8)
            def _(i):
                v = buf[a, slot, i >> 5, pl.ds((i & 31) * _L, _L)]
                b = (v * 255.0).astype(jnp.int32)
                bu = jnp.minimum(plsc.bitcast(b, jnp.uint32), jnp.uint32(255))
                plsc.addupdate_scatter(
                    hist, [plsc.bitcast(bu, jnp.int32) + base_vec], ones)

    for p in range(3):
        issue(jnp.int32(p), p)

    def jbody(jj, c):
        j0 = jj * 4
        for p in range(4):
            nxt = j0 + p + 3
            @pl.when(nxt < _CHUNKS_PER_W)
            def _(nxt=nxt, p=p):
                issue(nxt, (p + 3) % 4)
            wait(p)
            process(j0 + p, p)
        return c

    lax.fori_loop(0, _CHUNKS_PER_W // 4, jbody, 0)

    for a in (0, 1):
        for l in (0, 1):
            base = a * (2 * _L * 256) + l * (_L * 256)

            def gbody(g, c, base=base, a=a, l=l):
                off = g * _L
                acc = hist[pl.ds(base + off, _L)]
                for lane in range(1, _L):
                    acc = acc + hist[pl.ds(base + lane * 256 + off, _L)]
                stage[pl.ds((a * 2 + l) * 256 + off, _L)] = acc
                return c

            lax.fori_loop(0, 256 // _L, gbody, 0)

    for a in (0, 1):
        for l in (0, 1):
            dst_off = pl.multiple_of(a * (_NW * 2 * 256) + wid * 512 + l * 256, 256)
            pltpu.sync_copy(stage.at[pl.ds((a * 2 + l) * 256, 256)],
                            out_hbm.at[pl.ds(dst_off, 256)])


@functools.cache
def _sc_hist():
    return pl.kernel(
        _sc_body,
        out_type=jax.ShapeDtypeStruct((_OUT,), jnp.float32),
        mesh=plsc.VectorSubcoreMesh(
            core_axis_name="c", subcore_axis_name="s",
            num_cores=_NC, num_subcores=_NS),
        compiler_params=pltpu.CompilerParams(
            needs_layout_passes=False, use_tc_tiling_on_sc=True),
        scratch_types=[
            pltpu.VMEM((2, 4, _CHUNK // 512, 512), jnp.float32),
            pltpu.VMEM((_HIST,), jnp.float32),
            pltpu.VMEM((2 * 2 * 256,), jnp.float32),
            pltpu.SemaphoreType.DMA,
            pltpu.SemaphoreType.DMA,
            pltpu.SemaphoreType.DMA,
            pltpu.SemaphoreType.DMA,
        ],
    )


def _chi_body(s10, s11, s12, s13, s20, s21, s22, s23, out_ref):
    bias = 1e-10

    def f(c1, c2):
        h1 = c1 / 786432.0
        h2 = c2 / 786432.0
        d = h1 - h2
        return d * d / (h1 + h2 + bias)

    t10 = s10[...]
    t11 = s11[...] + s12[...]
    t12 = s13[...]
    t20 = s20[...]
    t21 = s21[...] + s22[...]
    t22 = s23[...]
    acc = f(t10, t20) + f(t11, t21) + f(t12, t22)
    out_ref[0, 0] = jnp.sum(acc) / 16.0


def kernel(hist1, hist2):
    counts = _sc_hist()(hist1.reshape(-1, 512), hist2.reshape(-1, 512))
    s = counts.reshape(2, 16, 4, 256)
    out = pl.pallas_call(
        _chi_body,
        out_shape=jax.ShapeDtypeStruct((1, 1), jnp.float32),
        out_specs=pl.BlockSpec(memory_space=pltpu.SMEM),
    )(s[0, :, 0], s[0, :, 1], s[0, :, 2], s[0, :, 3],
      s[1, :, 0], s[1, :, 1], s[1, :, 2], s[1, :, 3])
    return out[0, 0]


# speedup vs baseline: 1.0207x; 1.0207x over previous
import functools

import jax
import jax.numpy as jnp
from jax import lax
from jax.experimental import pallas as pl
from jax.experimental.pallas import tpu as pltpu
from jax.experimental.pallas import tpu_sc as plsc

_NC = 2
_NS = 16
_NW = _NC * _NS
_L = 16

_ROWS = 48
_ROW = 512 * 512
_TOTAL = _ROWS * _ROW
_CHUNK = 8192
_CHUNKS_PER_W = _TOTAL // _NW // _CHUNK
_PER_W = _TOTAL // _NW
_HIST = 2 * 2 * _L * 256
_OUT = 2 * _NW * 2 * 256


def _sc_body(h1_hbm, h2_hbm, out_hbm, buf, hist, stage, sem0, sem1):
    wid = lax.axis_index("s") * _NC + lax.axis_index("c")
    base_w = wid * _PER_W
    sems = (sem0, sem1)
    lane_vec = lax.iota(jnp.int32, _L) * 256
    ones = jnp.full((_L,), 1.0, jnp.float32)
    zeros = jnp.zeros((_L,), jnp.float32)

    @plsc.parallel_loop(0, _HIST, _L, unroll=4)
    def _(i):
        hist[pl.ds(i, _L)] = zeros

    def issue(j, slot):
        row0 = pl.multiple_of((base_w + j * _CHUNK) // 512, _CHUNK // 512)
        pltpu.async_copy(h1_hbm.at[pl.ds(row0, _CHUNK // 512)], buf.at[0, slot],
                         sems[slot])
        pltpu.async_copy(h2_hbm.at[pl.ds(row0, _CHUNK // 512)], buf.at[1, slot],
                         sems[slot])

    def wait(slot):
        pltpu.make_async_copy(
            h1_hbm.at[pl.ds(0, _CHUNK // 512)], buf.at[0, slot], sems[slot]).wait()
        pltpu.make_async_copy(
            h2_hbm.at[pl.ds(0, _CHUNK // 512)], buf.at[1, slot], sems[slot]).wait()

    def process(j, slot):
        cpr = _ROW // _CHUNK
        l = (j + (cpr // 2) * (wid % 2)) // cpr
        for a in (0, 1):
            base_vec = lane_vec + a * (2 * _L * 256) + l * (_L * 256)

            @plsc.parallel_loop(0, _CHUNK // _L, 1, unroll=16)
            def _(i):
                v = buf[a, slot, i >> 5, pl.ds((i & 31) * _L, _L)]
                b = (v * 255.0).astype(jnp.int32)
                bu = jnp.minimum(plsc.bitcast(b, jnp.uint32), jnp.uint32(255))
                plsc.addupdate_scatter(
                    hist, [plsc.bitcast(bu, jnp.int32) + base_vec], ones)

    issue(jnp.int32(0), 0)

    def jbody(jj, c):
        j0 = jj * 2
        issue(j0 + 1, 1)
        wait(0)
        process(j0, 0)

        @pl.when(j0 + 2 < _CHUNKS_PER_W)
        def _():
            issue(j0 + 2, 0)

        wait(1)
        process(j0 + 1, 1)
        return c

    lax.fori_loop(0, _CHUNKS_PER_W // 2, jbody, 0)

    for a in (0, 1):
        for l in (0, 1):
            base = a * (2 * _L * 256) + l * (_L * 256)

            def gbody(g, c, base=base, a=a, l=l):
                off = g * _L
                acc = hist[pl.ds(base + off, _L)]
                for lane in range(1, _L):
                    acc = acc + hist[pl.ds(base + lane * 256 + off, _L)]
                stage[pl.ds((a * 2 + l) * 256 + off, _L)] = acc
                return c

            lax.fori_loop(0, 256 // _L, gbody, 0)

    for a in (0, 1):
        for l in (0, 1):
            dst_off = pl.multiple_of(a * (_NW * 2 * 256) + wid * 512 + l * 256, 256)
            pltpu.sync_copy(stage.at[pl.ds((a * 2 + l) * 256, 256)],
                            out_hbm.at[pl.ds(dst_off, 256)])


@functools.cache
def _sc_hist():
    return pl.kernel(
        _sc_body,
        out_type=jax.ShapeDtypeStruct((_OUT,), jnp.float32),
        mesh=plsc.VectorSubcoreMesh(
            core_axis_name="c", subcore_axis_name="s",
            num_cores=_NC, num_subcores=_NS),
        compiler_params=pltpu.CompilerParams(
            needs_layout_passes=False, use_tc_tiling_on_sc=True),
        scratch_types=[
            pltpu.VMEM((2, 2, _CHUNK // 512, 512), jnp.float32),
            pltpu.VMEM((_HIST,), jnp.float32),
            pltpu.VMEM((2 * 2 * 256,), jnp.float32),
            pltpu.SemaphoreType.DMA,
            pltpu.SemaphoreType.DMA,
        ],
    )


def _chi_body(s10, s11, s12, s13, s20, s21, s22, s23, out_ref):
    bias = 1e-10

    def f(c1, c2):
        h1 = c1 / 786432.0
        h2 = c2 / 786432.0
        d = h1 - h2
        return d * d / (h1 + h2 + bias)

    t10 = s10[...]
    t11 = s11[...] + s12[...]
    t12 = s13[...]
    t20 = s20[...]
    t21 = s21[...] + s22[...]
    t22 = s23[...]
    acc = f(t10, t20) + f(t11, t21) + f(t12, t22)
    out_ref[0, 0] = jnp.sum(acc) / 16.0


def kernel(hist1, hist2):
    counts = _sc_hist()(hist1.reshape(-1, 512), hist2.reshape(-1, 512))
    s = counts.reshape(2, 16, 4, 256)
    out = pl.pallas_call(
        _chi_body,
        out_shape=jax.ShapeDtypeStruct((1, 1), jnp.float32),
        out_specs=pl.BlockSpec(memory_space=pltpu.SMEM),
    )(s[0, :, 0], s[0, :, 1], s[0, :, 2], s[0, :, 3],
      s[1, :, 0], s[1, :, 1], s[1, :, 2], s[1, :, 3])
    return out[0, 0]
